# baseline (speedup 1.0000x reference)
import functools

import jax
import jax.numpy as jnp
from jax import lax
from jax.experimental import pallas as pl
from jax.experimental.pallas import tpu as pltpu
from jax.experimental.pallas import tpu_sc as plsc

B, H, W, E = 256, 25, 25, 64
P = H * W
NPROP = 9
OFFS = (0, 20, 30, 36, 46, 56, 62, 68, 74)
RSTRIDE = 33
ZROW = 82 * RSTRIDE
COPY = 83 * RSTRIDE
NCOPY = 8
TALLOC = 21920
QP = 16
QC = 32
QW = QC * P
QPAD = QW + 16

NC, NS = 2, 16
NW = NC * NS
BPW = B // NW
NG = (P + 15) // 16
IDXW = BPW * P
IDXPAD = 5024
MSKW = 8 * P
MSKPAD = 5024

_mesh = plsc.VectorSubcoreMesh(core_axis_name="c", subcore_axis_name="s")


@functools.partial(
    pl.kernel,
    out_type=jax.ShapeDtypeStruct((B, 4 * QW), jnp.float32),
    mesh=_mesh,
    scratch_types=[
        pltpu.VMEM((NPROP, IDXPAD), jnp.int32),
        pltpu.VMEM((MSKPAD,), jnp.float32),
        pltpu.VMEM((MSKPAD,), jnp.float32),
        pltpu.VMEM((TALLOC,), jnp.int32),
        pltpu.VMEM((QPAD,), jnp.float32),
        pltpu.VMEM((QPAD,), jnp.float32),
        pltpu.SemaphoreType.DMA,
        pltpu.SemaphoreType.DMA,
        pltpu.SemaphoreType.DMA,
    ],
    compiler_params=pltpu.CompilerParams(
        use_tc_tiling_on_sc=False, needs_layout_passes=False),
)
def _sc_embed(i0, i1, i2, i3, i4, i5, i6, i7, i8, mask_hbm, w_hbm, out_hbm,
              idx_v, mask_v0, mask_v1, tbl_v, q0_v, q1_v, sem0, sem1, semi):
    idx_hbms = (i0, i1, i2, i3, i4, i5, i6, i7, i8)
    wid = lax.axis_index("s") * NC + lax.axis_index("c")
    bufs = (q0_v, q1_v)
    sems = (sem0, sem1)

    lane = lax.broadcasted_iota(jnp.int32, (16,), 0)
    rep = (lane % NCOPY) * COPY
    zero16 = jnp.zeros((16,), jnp.float32)
    zero16i = jnp.zeros((16,), jnp.int32)

    for i in range(NPROP):
        idx_v[i, pl.ds(IDXW - 24, 16)] = zero16i
        idx_v[i, pl.ds(IDXW - 8, 16)] = zero16i
        idx_v[i, pl.ds(IDXW + 8, 16)] = zero16i
    for mv in (mask_v0, mask_v1):
        mv[pl.ds(MSKW - 24, 16)] = zero16
        mv[pl.ds(MSKW - 8, 16)] = zero16
        mv[pl.ds(MSKW + 8, 16)] = zero16

    pltpu.sync_copy(w_hbm, tbl_v)
    for i in range(NPROP):
        pltpu.async_copy(
            idx_hbms[i].at[pl.ds(wid * IDXW, IDXW)],
            idx_v.at[i, pl.ds(0, IDXW)], semi)

    def fetch_mask(b, mv):
        pltpu.async_copy(
            mask_hbm.at[pl.ds(b * MSKW, MSKW)], mv.at[pl.ds(0, MSKW)], semi)

    def drain_idx():
        for i in range(NPROP):
            pltpu.make_async_copy(
                idx_hbms[i].at[pl.ds(0, IDXW)],
                idx_v.at[i, pl.ds(0, IDXW)], semi).wait()

    def drain_mask():
        pltpu.make_async_copy(
            mask_hbm.at[pl.ds(0, MSKW)],
            mask_v0.at[pl.ds(0, MSKW)], semi).wait()

    def drain_out(sem):
        pltpu.make_async_copy(
            out_hbm.at[0, pl.ds(0, QW)], q0_v.at[pl.ds(0, QW)], sem).wait()

    drain_idx()
    fetch_mask(wid * BPW, mask_v0)

    def one_batch(t, carry, mask_v=None):
        b = wid * BPW + t
        pxbase = t * P

        for q in range(4):
            buf, sem = bufs[q % 2], sems[q % 2]
            if q < 2:
                @pl.when(t > 0)
                def _():
                    drain_out(sem)
            else:
                drain_out(sem)

            terr = q >= 2
            cbase = (q - 2) * QP if terr else q * QP

            def one_group(g, carry2, terr=terr, cbase=cbase, buf=buf):
                px = pl.multiple_of(g * 16, 16)
                valid = lane < (P - px)
                ivec = lane + (pxbase + px)
                mvec = lane + px
                if terr:
                    r8 = (plsc.load_gather(idx_v.at[8], [ivec]) * RSTRIDE
                          + (OFFS[8] * RSTRIDE + cbase)) + rep
                    for c in range(QP):
                        gw = plsc.load_gather(tbl_v, [r8 + c if c else r8])
                        lo, hi = plsc.unpack(
                            plsc.bitcast(gw, jnp.bfloat16),
                            format=plsc.PackFormat.INTERLEAVED,
                            preferred_element_type=jnp.float32)
                        plsc.store_compressed(
                            buf.at[pl.ds((2 * c) * P + px, 16)], lo,
                            mask=valid)
                        plsc.store_compressed(
                            buf.at[pl.ds((2 * c + 1) * P + px, 16)], hi,
                            mask=valid)
                else:
                    rows = []
                    for i in range(8):
                        ri = (plsc.load_gather(idx_v.at[i], [ivec]) * RSTRIDE
                              + (OFFS[i] * RSTRIDE + cbase))
                        mi = plsc.load_gather(mask_v, [mvec + i * P]) > 0.0
                        rows.append(jnp.where(mi, ri, ZROW + cbase) + rep)
                    for c in range(QP):
                        g8 = [plsc.bitcast(
                                  plsc.load_gather(
                                      tbl_v,
                                      [rows[i] + c if c else rows[i]]),
                                  jnp.bfloat16)
                              for i in range(8)]
                        acc = (((g8[0] + g8[1]) + (g8[2] + g8[3]))
                               + ((g8[4] + g8[5]) + (g8[6] + g8[7])))
                        lo, hi = plsc.unpack(
                            acc, format=plsc.PackFormat.INTERLEAVED,
                            preferred_element_type=jnp.float32)
                        plsc.store_compressed(
                            buf.at[pl.ds((2 * c) * P + px, 16)], lo,
                            mask=valid)
                        plsc.store_compressed(
                            buf.at[pl.ds((2 * c + 1) * P + px, 16)], hi,
                            mask=valid)
                return carry2

            lax.fori_loop(0, NG, one_group, 0, unroll=False)
            pltpu.async_copy(
                buf.at[pl.ds(0, QW)],
                out_hbm.at[b, pl.ds(q * QW, QW)], sem)
        return carry

    def one_pair(t2, carry):
        t_even = t2 * 2
        drain_mask()
        fetch_mask(wid * BPW + t_even + 1, mask_v1)
        one_batch(t_even, carry, mask_v=mask_v0)
        drain_mask()

        @pl.when(t_even + 2 < BPW)
        def _():
            fetch_mask(wid * BPW + t_even + 2, mask_v0)
        one_batch(t_even + 1, carry, mask_v=mask_v1)
        return carry

    lax.fori_loop(0, BPW // 2, one_pair, 0, unroll=False)
    drain_out(sem0)
    drain_out(sem1)


def kernel(prop_types, hut_colors, hut_rotations, tree_types, plant_types,
           windmill_rotations, tower_rotations, tent_rotations, terrain,
           nonempty_property_mask, weight):
    props = [prop_types, hut_colors, hut_rotations, tree_types, plant_types,
             windmill_rotations, tower_rotations, tent_rotations, terrain]
    idx = [p.reshape(B * P).astype(jnp.int32) for p in props]
    mask = nonempty_property_mask.reshape(B * 8 * P).astype(jnp.float32)
    wb = weight.astype(jnp.bfloat16)
    lo16 = lax.bitcast_convert_type(wb[:, 0::2], jnp.uint16).astype(jnp.uint32)
    hi16 = lax.bitcast_convert_type(wb[:, 1::2], jnp.uint16).astype(jnp.uint32)
    wpk = (lo16 | (hi16 << 16)).astype(jnp.int32)
    wpk = jnp.pad(wpk, ((0, 1), (0, RSTRIDE - E // 2))).reshape(-1)
    wpk = jnp.tile(wpk, NCOPY)
    wpad = jnp.pad(wpk, (0, TALLOC - wpk.shape[0]))
    out = _sc_embed(*idx, mask, wpad)
    return out.reshape(B, 2 * E, H, W)


# speedup vs baseline: 1.0078x; 1.0078x over previous
import functools

import jax
import jax.numpy as jnp
from jax import lax
from jax.experimental import pallas as pl
from jax.experimental.pallas import tpu as pltpu
from jax.experimental.pallas import tpu_sc as plsc

B, H, W, E = 256, 25, 25, 64
P = H * W
NPROP = 9
OFFS = (0, 20, 30, 36, 46, 56, 62, 68, 74)
RSTRIDE = 33
ZROW = 82 * RSTRIDE
COPY = 83 * RSTRIDE
NCOPY = 8
TALLOC = 21920
QP = 16
QC = 32
QW = QC * P
QPAD = QW + 16

NC, NS = 2, 16
NW = NC * NS
BPW = B // NW
NG = (P + 15) // 16
IDXW = BPW * P
IDXPAD = 5024
MSKW = 8 * P
MSKPAD = 5024

_mesh = plsc.VectorSubcoreMesh(core_axis_name="c", subcore_axis_name="s")


@functools.partial(
    pl.kernel,
    out_type=jax.ShapeDtypeStruct((B, 4 * QW), jnp.float32),
    mesh=_mesh,
    scratch_types=[
        pltpu.VMEM((NPROP, IDXPAD), jnp.int32),
        pltpu.VMEM((2 * MSKPAD,), jnp.float32),
        pltpu.VMEM((TALLOC,), jnp.int32),
        pltpu.VMEM((QPAD,), jnp.float32),
        pltpu.VMEM((QPAD,), jnp.float32),
        pltpu.SemaphoreType.DMA,
        pltpu.SemaphoreType.DMA,
        pltpu.SemaphoreType.DMA,
    ],
    compiler_params=pltpu.CompilerParams(
        use_tc_tiling_on_sc=False, needs_layout_passes=False),
)
def _sc_embed(i0, i1, i2, i3, i4, i5, i6, i7, i8, mask_hbm, w_hbm, out_hbm,
              idx_v, mask_v, tbl_v, q0_v, q1_v, sem0, sem1, semi):
    idx_hbms = (i0, i1, i2, i3, i4, i5, i6, i7, i8)
    wid = lax.axis_index("s") * NC + lax.axis_index("c")
    bufs = (q0_v, q1_v)
    sems = (sem0, sem1)

    lane = lax.broadcasted_iota(jnp.int32, (16,), 0)
    rep = (lane % NCOPY) * COPY
    zero16 = jnp.zeros((16,), jnp.float32)
    zero16i = jnp.zeros((16,), jnp.int32)

    for i in range(NPROP):
        idx_v[i, pl.ds(IDXW - 24, 16)] = zero16i
        idx_v[i, pl.ds(IDXW - 8, 16)] = zero16i
        idx_v[i, pl.ds(IDXW + 8, 16)] = zero16i
    for half in (0, MSKPAD):
        mask_v[pl.ds(half + MSKW - 24, 16)] = zero16
        mask_v[pl.ds(half + MSKW - 8, 16)] = zero16
        mask_v[pl.ds(half + MSKW + 8, 16)] = zero16

    pltpu.sync_copy(w_hbm, tbl_v)
    for i in range(NPROP):
        pltpu.async_copy(
            idx_hbms[i].at[pl.ds(wid * IDXW, IDXW)],
            idx_v.at[i, pl.ds(0, IDXW)], semi)

    def fetch_mask(b, half):
        pltpu.async_copy(
            mask_hbm.at[pl.ds(b * MSKW, MSKW)],
            mask_v.at[pl.ds(pl.multiple_of(half * MSKPAD, 8), MSKW)], semi)

    def drain_idx():
        for i in range(NPROP):
            pltpu.make_async_copy(
                idx_hbms[i].at[pl.ds(0, IDXW)],
                idx_v.at[i, pl.ds(0, IDXW)], semi).wait()

    def drain_mask():
        pltpu.make_async_copy(
            mask_hbm.at[pl.ds(0, MSKW)],
            mask_v.at[pl.ds(0, MSKW)], semi).wait()

    def drain_out(sem):
        pltpu.make_async_copy(
            out_hbm.at[0, pl.ds(0, QW)], q0_v.at[pl.ds(0, QW)], sem).wait()

    drain_idx()
    fetch_mask(wid * BPW, 0)

    def one_batch(t, carry):
        b = wid * BPW + t
        pxbase = t * P
        moff = (t % 2) * MSKPAD
        drain_mask()

        @pl.when(t + 1 < BPW)
        def _():
            fetch_mask(b + 1, (t + 1) % 2)

        for q in range(4):
            buf, sem = bufs[q % 2], sems[q % 2]
            if q < 2:
                @pl.when(t > 0)
                def _():
                    drain_out(sem)
            else:
                drain_out(sem)

            terr = q >= 2
            cbase = (q - 2) * QP if terr else q * QP

            def do_group(px, terr=terr, cbase=cbase, buf=buf):
                valid = lane < (P - px)
                ivec = lane + (pxbase + px)
                mvec = lane + (moff + px)
                if terr:
                    r8 = (plsc.load_gather(idx_v.at[8], [ivec]) * RSTRIDE
                          + (OFFS[8] * RSTRIDE + cbase)) + rep
                    for c in range(QP):
                        gw = plsc.load_gather(tbl_v, [r8 + c if c else r8])
                        lo, hi = plsc.unpack(
                            plsc.bitcast(gw, jnp.bfloat16),
                            format=plsc.PackFormat.INTERLEAVED,
                            preferred_element_type=jnp.float32)
                        plsc.store_compressed(
                            buf.at[pl.ds((2 * c) * P + px, 16)], lo,
                            mask=valid)
                        plsc.store_compressed(
                            buf.at[pl.ds((2 * c + 1) * P + px, 16)], hi,
                            mask=valid)
                else:
                    rows = []
                    for i in range(8):
                        ri = (plsc.load_gather(idx_v.at[i], [ivec]) * RSTRIDE
                              + (OFFS[i] * RSTRIDE + cbase))
                        mi = plsc.load_gather(mask_v, [mvec + i * P]) > 0.0
                        rows.append(jnp.where(mi, ri, ZROW + cbase) + rep)
                    for c in range(QP):
                        g8 = [plsc.bitcast(
                                  plsc.load_gather(
                                      tbl_v,
                                      [rows[i] + c if c else rows[i]]),
                                  jnp.bfloat16)
                              for i in range(8)]
                        acc = (((g8[0] + g8[1]) + (g8[2] + g8[3]))
                               + ((g8[4] + g8[5]) + (g8[6] + g8[7])))
                        lo, hi = plsc.unpack(
                            acc, format=plsc.PackFormat.INTERLEAVED,
                            preferred_element_type=jnp.float32)
                        plsc.store_compressed(
                            buf.at[pl.ds((2 * c) * P + px, 16)], lo,
                            mask=valid)
                        plsc.store_compressed(
                            buf.at[pl.ds((2 * c + 1) * P + px, 16)], hi,
                            mask=valid)

            def one_group2(g2, carry2):
                px = pl.multiple_of(g2 * 32, 16)
                do_group(px)
                do_group(px + 16)
                return carry2

            lax.fori_loop(0, NG // 2, one_group2, 0, unroll=False)
            pltpu.async_copy(
                buf.at[pl.ds(0, QW)],
                out_hbm.at[b, pl.ds(q * QW, QW)], sem)
        return carry

    lax.fori_loop(0, BPW, one_batch, 0, unroll=False)
    drain_out(sem0)
    drain_out(sem1)


def kernel(prop_types, hut_colors, hut_rotations, tree_types, plant_types,
           windmill_rotations, tower_rotations, tent_rotations, terrain,
           nonempty_property_mask, weight):
    props = [prop_types, hut_colors, hut_rotations, tree_types, plant_types,
             windmill_rotations, tower_rotations, tent_rotations, terrain]
    idx = [p.reshape(B * P).astype(jnp.int32) for p in props]
    mask = nonempty_property_mask.reshape(B * 8 * P).astype(jnp.float32)
    *idx, mask = lax.optimization_barrier((*idx, mask))
    wb = weight.astype(jnp.bfloat16)
    lo16 = lax.bitcast_convert_type(wb[:, 0::2], jnp.uint16).astype(jnp.uint32)
    hi16 = lax.bitcast_convert_type(wb[:, 1::2], jnp.uint16).astype(jnp.uint32)
    wpk = (lo16 | (hi16 << 16)).astype(jnp.int32)
    wpk = jnp.pad(wpk, ((0, 1), (0, RSTRIDE - E // 2))).reshape(-1)
    wpk = jnp.tile(wpk, NCOPY)
    wpad = jnp.pad(wpk, (0, TALLOC - wpk.shape[0]))
    out = _sc_embed(*idx, mask, wpad)
    return out.reshape(B, 2 * E, H, W)


# speedup vs baseline: 1.0873x; 1.0789x over previous
import functools

import jax
import jax.numpy as jnp
from jax import lax
from jax.experimental import pallas as pl
from jax.experimental.pallas import tpu as pltpu
from jax.experimental.pallas import tpu_sc as plsc

B, H, W, E = 256, 25, 25, 64
P = H * W
PP = 640
NPROP = 9
OFFS = (0, 20, 30, 36, 46, 56, 62, 68, 74)
RSTRIDE = 33
ZROW = 82 * RSTRIDE
COPY = 83 * RSTRIDE
NCOPY = 8
TALLOC = 21920
QP = 16
QC = 32
QW = QC * P
QPAD = QW + 16

NC, NS = 2, 16
NW = NC * NS
BPW = B // NW
NG = PP // 16

_mesh = plsc.VectorSubcoreMesh(core_axis_name="c", subcore_axis_name="s")


@functools.partial(
    pl.kernel,
    out_type=jax.ShapeDtypeStruct((B, 4 * QW), jnp.float32),
    mesh=_mesh,
    scratch_types=[
        pltpu.VMEM((NPROP, PP), jnp.int32),
        pltpu.VMEM((NPROP, PP), jnp.int32),
        pltpu.VMEM((8, PP), jnp.float32),
        pltpu.VMEM((8, PP), jnp.float32),
        pltpu.VMEM((TALLOC,), jnp.int32),
        pltpu.VMEM((QPAD,), jnp.float32),
        pltpu.VMEM((QPAD,), jnp.float32),
        pltpu.SemaphoreType.DMA,
        pltpu.SemaphoreType.DMA,
        pltpu.SemaphoreType.DMA,
    ],
    compiler_params=pltpu.CompilerParams(
        use_tc_tiling_on_sc=False, needs_layout_passes=False),
)
def _sc_embed(idx_hbm, mask_hbm, w_hbm, out_hbm,
              idx_v0, idx_v1, mask_v0, mask_v1, tbl_v, q0_v, q1_v,
              sem0, sem1, semi):
    wid = lax.axis_index("s") * NC + lax.axis_index("c")
    bufs = (q0_v, q1_v)
    sems = (sem0, sem1)

    pltpu.sync_copy(w_hbm, tbl_v)
    lane = lax.broadcasted_iota(jnp.int32, (16,), 0)
    rep = (lane % NCOPY) * COPY

    def drain(sem):
        pltpu.make_async_copy(
            out_hbm.at[0, pl.ds(0, QW)], q0_v.at[pl.ds(0, QW)], sem).wait()

    def fetch_inputs(b, iv, mv):
        pltpu.async_copy(idx_hbm.at[b], iv, semi)
        pltpu.async_copy(mask_hbm.at[b], mv, semi)

    def drain_inputs():
        pltpu.make_async_copy(idx_hbm.at[0], idx_v0, semi).wait()
        pltpu.make_async_copy(mask_hbm.at[0], mask_v0, semi).wait()

    fetch_inputs(wid * BPW, idx_v0, mask_v0)

    def one_batch(t, carry, idx_v=None, mask_v=None):
        b = wid * BPW + t

        for q in range(4):
            buf, sem = bufs[q % 2], sems[q % 2]
            if q < 2:
                @pl.when(t > 0)
                def _():
                    drain(sem)
            else:
                drain(sem)

            terr = q >= 2
            cbase = (q - 2) * QP if terr else q * QP

            def one_group(g, carry2, terr=terr, cbase=cbase, buf=buf):
                px = pl.multiple_of(g * 16, 16)
                valid = lane < (P - px)
                if terr:
                    r8 = (idx_v[8, pl.ds(px, 16)] * RSTRIDE
                          + (OFFS[8] * RSTRIDE + cbase)) + rep
                    for c in range(QP):
                        gw = plsc.load_gather(tbl_v, [r8 + c if c else r8])
                        lo, hi = plsc.unpack(
                            plsc.bitcast(gw, jnp.bfloat16),
                            format=plsc.PackFormat.INTERLEAVED,
                            preferred_element_type=jnp.float32)
                        plsc.store_compressed(
                            buf.at[pl.ds((2 * c) * P + px, 16)], lo,
                            mask=valid)
                        plsc.store_compressed(
                            buf.at[pl.ds((2 * c + 1) * P + px, 16)], hi,
                            mask=valid)
                else:
                    rows = []
                    for i in range(8):
                        ri = (idx_v[i, pl.ds(px, 16)] * RSTRIDE
                              + (OFFS[i] * RSTRIDE + cbase))
                        mi = mask_v[i, pl.ds(px, 16)] > 0.0
                        rows.append(jnp.where(mi, ri, ZROW + cbase) + rep)
                    for c in range(QP):
                        g8 = [plsc.bitcast(
                                  plsc.load_gather(
                                      tbl_v,
                                      [rows[i] + c if c else rows[i]]),
                                  jnp.bfloat16)
                              for i in range(8)]
                        acc = (((g8[0] + g8[1]) + (g8[2] + g8[3]))
                               + ((g8[4] + g8[5]) + (g8[6] + g8[7])))
                        lo, hi = plsc.unpack(
                            acc, format=plsc.PackFormat.INTERLEAVED,
                            preferred_element_type=jnp.float32)
                        plsc.store_compressed(
                            buf.at[pl.ds((2 * c) * P + px, 16)], lo,
                            mask=valid)
                        plsc.store_compressed(
                            buf.at[pl.ds((2 * c + 1) * P + px, 16)], hi,
                            mask=valid)
                return carry2

            lax.fori_loop(0, NG, one_group, 0, unroll=False)
            pltpu.async_copy(
                buf.at[pl.ds(0, QW)],
                out_hbm.at[b, pl.ds(q * QW, QW)], sem)
        return carry

    def one_pair(t2, carry):
        t_even = t2 * 2
        drain_inputs()
        fetch_inputs(wid * BPW + t_even + 1, idx_v1, mask_v1)
        one_batch(t_even, carry, idx_v=idx_v0, mask_v=mask_v0)
        drain_inputs()

        @pl.when(t_even + 2 < BPW)
        def _():
            fetch_inputs(wid * BPW + t_even + 2, idx_v0, mask_v0)
        one_batch(t_even + 1, carry, idx_v=idx_v1, mask_v=mask_v1)
        return carry

    lax.fori_loop(0, BPW // 2, one_pair, 0, unroll=False)
    drain(sem0)
    drain(sem1)


def kernel(prop_types, hut_colors, hut_rotations, tree_types, plant_types,
           windmill_rotations, tower_rotations, tent_rotations, terrain,
           nonempty_property_mask, weight):
    props = [prop_types, hut_colors, hut_rotations, tree_types, plant_types,
             windmill_rotations, tower_rotations, tent_rotations, terrain]
    idx = jnp.stack(
        [p.reshape(B, P).astype(jnp.int32) for p in props], axis=1)
    idx = jnp.pad(idx, ((0, 0), (0, 0), (0, PP - P)))
    mask = jnp.pad(
        nonempty_property_mask.reshape(B, 8, P).astype(jnp.float32),
        ((0, 0), (0, 0), (0, PP - P)))
    wb = weight.astype(jnp.bfloat16)
    lo16 = lax.bitcast_convert_type(wb[:, 0::2], jnp.uint16).astype(jnp.uint32)
    hi16 = lax.bitcast_convert_type(wb[:, 1::2], jnp.uint16).astype(jnp.uint32)
    wpk = (lo16 | (hi16 << 16)).astype(jnp.int32)
    wpk = jnp.pad(wpk, ((0, 1), (0, RSTRIDE - E // 2))).reshape(-1)
    wpk = jnp.tile(wpk, NCOPY)
    wpad = jnp.pad(wpk, (0, TALLOC - wpk.shape[0]))
    out = _sc_embed(idx, mask, wpad)
    return out.reshape(B, 2 * E, H, W)


# speedup vs baseline: 1.0978x; 1.0097x over previous
import functools

import jax
import jax.numpy as jnp
from jax import lax
from jax.experimental import pallas as pl
from jax.experimental.pallas import tpu as pltpu
from jax.experimental.pallas import tpu_sc as plsc

B, H, W, E = 256, 25, 25, 64
P = H * W
PP = 640
NPROP = 9
OFFS = (0, 20, 30, 36, 46, 56, 62, 68, 74)
RSTRIDE = 33
ZROW = 82 * RSTRIDE
COPY = 83 * RSTRIDE
NCOPY = 16
TALLOC = 43888
QP = 16
QC = 32
QW = QC * P
QPAD = QW + 16

NC, NS = 2, 16
NW = NC * NS
BPW = B // NW
NG = PP // 16

_mesh = plsc.VectorSubcoreMesh(core_axis_name="c", subcore_axis_name="s")


@functools.partial(
    pl.kernel,
    out_type=jax.ShapeDtypeStruct((B, 4 * QW), jnp.float32),
    mesh=_mesh,
    scratch_types=[
        pltpu.VMEM((NPROP, PP), jnp.int32),
        pltpu.VMEM((NPROP, PP), jnp.int32),
        pltpu.VMEM((8, PP), jnp.float32),
        pltpu.VMEM((8, PP), jnp.float32),
        pltpu.VMEM((TALLOC,), jnp.int32),
        pltpu.VMEM((QPAD,), jnp.float32),
        pltpu.VMEM((QPAD,), jnp.float32),
        pltpu.SemaphoreType.DMA,
        pltpu.SemaphoreType.DMA,
        pltpu.SemaphoreType.DMA,
    ],
    compiler_params=pltpu.CompilerParams(
        use_tc_tiling_on_sc=False, needs_layout_passes=False),
)
def _sc_embed(idx_hbm, mask_hbm, w_hbm, out_hbm,
              idx_v0, idx_v1, mask_v0, mask_v1, tbl_v, q0_v, q1_v,
              sem0, sem1, semi):
    wid = lax.axis_index("s") * NC + lax.axis_index("c")
    bufs = (q0_v, q1_v)
    sems = (sem0, sem1)

    pltpu.sync_copy(w_hbm, tbl_v)
    lane = lax.broadcasted_iota(jnp.int32, (16,), 0)
    rep = (lane % NCOPY) * COPY

    def drain(sem):
        pltpu.make_async_copy(
            out_hbm.at[0, pl.ds(0, QW)], q0_v.at[pl.ds(0, QW)], sem).wait()

    def fetch_inputs(b, iv, mv):
        pltpu.async_copy(idx_hbm.at[b], iv, semi)
        pltpu.async_copy(mask_hbm.at[b], mv, semi)

    def drain_inputs():
        pltpu.make_async_copy(idx_hbm.at[0], idx_v0, semi).wait()
        pltpu.make_async_copy(mask_hbm.at[0], mask_v0, semi).wait()

    fetch_inputs(wid * BPW, idx_v0, mask_v0)

    def one_batch(t, carry, idx_v=None, mask_v=None):
        b = wid * BPW + t

        for q in range(4):
            buf, sem = bufs[q % 2], sems[q % 2]
            if q < 2:
                @pl.when(t > 0)
                def _():
                    drain(sem)
            else:
                drain(sem)

            terr = q >= 2
            cbase = (q - 2) * QP if terr else q * QP

            def one_group(g, carry2, terr=terr, cbase=cbase, buf=buf):
                px = pl.multiple_of(g * 16, 16)
                valid = lane < (P - px)
                if terr:
                    r8 = (idx_v[8, pl.ds(px, 16)] * RSTRIDE
                          + (OFFS[8] * RSTRIDE + cbase)) + rep
                    for c in range(QP):
                        gw = plsc.load_gather(tbl_v, [r8 + c if c else r8])
                        lo, hi = plsc.unpack(
                            plsc.bitcast(gw, jnp.bfloat16),
                            format=plsc.PackFormat.INTERLEAVED,
                            preferred_element_type=jnp.float32)
                        plsc.store_compressed(
                            buf.at[pl.ds((2 * c) * P + px, 16)], lo,
                            mask=valid)
                        plsc.store_compressed(
                            buf.at[pl.ds((2 * c + 1) * P + px, 16)], hi,
                            mask=valid)
                else:
                    rows = []
                    for i in range(8):
                        ri = (idx_v[i, pl.ds(px, 16)] * RSTRIDE
                              + (OFFS[i] * RSTRIDE + cbase))
                        mi = mask_v[i, pl.ds(px, 16)] > 0.0
                        rows.append(jnp.where(mi, ri, ZROW + cbase) + rep)
                    for c in range(QP):
                        g8 = [plsc.bitcast(
                                  plsc.load_gather(
                                      tbl_v,
                                      [rows[i] + c if c else rows[i]]),
                                  jnp.bfloat16)
                              for i in range(8)]
                        acc = (((g8[0] + g8[1]) + (g8[2] + g8[3]))
                               + ((g8[4] + g8[5]) + (g8[6] + g8[7])))
                        lo, hi = plsc.unpack(
                            acc, format=plsc.PackFormat.INTERLEAVED,
                            preferred_element_type=jnp.float32)
                        plsc.store_compressed(
                            buf.at[pl.ds((2 * c) * P + px, 16)], lo,
                            mask=valid)
                        plsc.store_compressed(
                            buf.at[pl.ds((2 * c + 1) * P + px, 16)], hi,
                            mask=valid)
                return carry2

            lax.fori_loop(0, NG, one_group, 0, unroll=False)
            pltpu.async_copy(
                buf.at[pl.ds(0, QW)],
                out_hbm.at[b, pl.ds(q * QW, QW)], sem)
        return carry

    def one_pair(t2, carry):
        t_even = t2 * 2
        drain_inputs()
        fetch_inputs(wid * BPW + t_even + 1, idx_v1, mask_v1)
        one_batch(t_even, carry, idx_v=idx_v0, mask_v=mask_v0)
        drain_inputs()

        @pl.when(t_even + 2 < BPW)
        def _():
            fetch_inputs(wid * BPW + t_even + 2, idx_v0, mask_v0)
        one_batch(t_even + 1, carry, idx_v=idx_v1, mask_v=mask_v1)
        return carry

    lax.fori_loop(0, BPW // 2, one_pair, 0, unroll=False)
    drain(sem0)
    drain(sem1)


def kernel(prop_types, hut_colors, hut_rotations, tree_types, plant_types,
           windmill_rotations, tower_rotations, tent_rotations, terrain,
           nonempty_property_mask, weight):
    props = [prop_types, hut_colors, hut_rotations, tree_types, plant_types,
             windmill_rotations, tower_rotations, tent_rotations, terrain]
    idx = jnp.stack(
        [p.reshape(B, P).astype(jnp.int32) for p in props], axis=1)
    idx = jnp.pad(idx, ((0, 0), (0, 0), (0, PP - P)))
    mask = jnp.pad(
        nonempty_property_mask.reshape(B, 8, P).astype(jnp.float32),
        ((0, 0), (0, 0), (0, PP - P)))
    wb = weight.astype(jnp.bfloat16)
    lo16 = lax.bitcast_convert_type(wb[:, 0::2], jnp.uint16).astype(jnp.uint32)
    hi16 = lax.bitcast_convert_type(wb[:, 1::2], jnp.uint16).astype(jnp.uint32)
    wpk = (lo16 | (hi16 << 16)).astype(jnp.int32)
    wpk = jnp.pad(wpk, ((0, 1), (0, RSTRIDE - E // 2))).reshape(-1)
    wpk = jnp.tile(wpk, NCOPY)
    wpad = jnp.pad(wpk, (0, TALLOC - wpk.shape[0]))
    out = _sc_embed(idx, mask, wpad)
    return out.reshape(B, 2 * E, H, W)
